# Initial kernel scaffold; baseline (speedup 1.0000x reference)
import jax
import jax.numpy as jnp
from jax.experimental import pallas as pl


def kernel(dep_x, dep_edge_index, dep_batch, boxes, labels, W_dep1, b_dep1, W_dep2, b_dep2, W_obj1, b_obj1, W_obj2, b_obj2, W_rel_a, W_rel_b, W_fus1, b_fus1, W_fus2, b_fus2):
    raise NotImplementedError("write your pallas kernel here")


# speedup vs baseline: 26.7205x; 26.7205x over previous
import functools

import jax
import jax.numpy as jnp
from jax import lax
from jax.experimental import pallas as pl
from jax.experimental.pallas import tpu as pltpu
from jax.experimental.pallas import tpu_sc as plsc

N = 10000
E = 320000
D = 128
BATCH = 128
NBOX = 36
PBOX = 40
NOBJ = 1601
HOBJ = 256
DREL = 64
NANS = 3129
HFUS = (2 * D + NANS) // 2
HFUS_P = 1792
NANS_P = 3200

NC = 2
NS = 16
NW = NC * NS
EPW = E // NW
CH = 125
K = EPW // CH
RQ = 624
TAIL = N - NS * RQ
HD = D // 2
LPW = (BATCH * NBOX) // NW
LCH = 2
LK = LPW // LCH


@functools.cache
def _build_sc_deg():
    mesh = plsc.VectorSubcoreMesh(core_axis_name="c", subcore_axis_name="s")
    return functools.partial(
        pl.kernel,
        out_type=jax.ShapeDtypeStruct((NC, N, 16), jnp.float32),
        mesh=mesh,
        scratch_types=[
            pltpu.VMEM((CH, K), jnp.int32),
            pltpu.VMEM((K, 16), jnp.float32),
            pltpu.VMEM_SHARED((N, 16), jnp.float32),
        ],
        compiler_params=pltpu.CompilerParams(use_tc_tiling_on_sc=False),
    )(_sc_deg_body)


def _sc_deg(dst, ones16, zeros16):
    return _build_sc_deg()(dst, ones16, zeros16)


def _sc_deg_body(dst_hbm, ones_hbm, zeros_hbm, out_hbm, idx_v, ones_v, acc_sh):
    c = lax.axis_index("c")
    s = lax.axis_index("s")
    w = c * NS + s
    my_rows = pl.ds(s * RQ, RQ)
    tail = pl.ds(NS * RQ, TAIL)
    pltpu.sync_copy(zeros_hbm, acc_sh.at[my_rows])

    @pl.when(s == NS - 1)
    def _():
        pltpu.sync_copy(zeros_hbm.at[pl.ds(0, TAIL)], acc_sh.at[tail])

    pltpu.sync_copy(ones_hbm, ones_v)
    pltpu.sync_copy(dst_hbm.at[w], idx_v)
    plsc.subcore_barrier()

    @pl.loop(0, CH)
    def _chunk(j):
        pltpu.sync_copy(ones_v, acc_sh.at[idx_v.at[j]], add=True)

    plsc.subcore_barrier()
    pltpu.sync_copy(acc_sh.at[my_rows], out_hbm.at[c].at[my_rows])

    @pl.when(s == NS - 1)
    def _():
        pltpu.sync_copy(acc_sh.at[tail], out_hbm.at[c].at[tail])


@functools.cache
def _build_sc_agg():
    mesh = plsc.VectorSubcoreMesh(core_axis_name="c", subcore_axis_name="s")
    return functools.partial(
        pl.kernel,
        out_type=jax.ShapeDtypeStruct((NC, N, HD), jnp.float32),
        mesh=mesh,
        scratch_types=[
            pltpu.VMEM((CH, K), jnp.int32),
            pltpu.VMEM((CH, K), jnp.int32),
            pltpu.VMEM((K, HD), jnp.float32),
            pltpu.VMEM((K, HD), jnp.float32),
            pltpu.VMEM_SHARED((N, HD), jnp.float32),
            pltpu.SemaphoreType.DMA,
            pltpu.SemaphoreType.DMA,
        ],
        compiler_params=pltpu.CompilerParams(use_tc_tiling_on_sc=False),
    )(_sc_agg_body)


def _sc_agg(hp_half, src, dst, zerosH):
    return _build_sc_agg()(hp_half, src, dst, zerosH)


def _sc_agg_body(h_hbm, src_hbm, dst_hbm, zeros_hbm, out_hbm,
                 idxs_v, idxd_v, rows0, rows1, acc_sh, sem0, sem1):
    c = lax.axis_index("c")
    s = lax.axis_index("s")
    w = c * NS + s
    my_rows = pl.ds(s * RQ, RQ)
    tail = pl.ds(NS * RQ, TAIL)
    last = s == NS - 1

    @pl.when(c == 0)
    def _():
        pltpu.sync_copy(h_hbm.at[my_rows], acc_sh.at[my_rows])

        @pl.when(last)
        def _():
            pltpu.sync_copy(h_hbm.at[tail], acc_sh.at[tail])

    @pl.when(c != 0)
    def _():
        pltpu.sync_copy(zeros_hbm, acc_sh.at[my_rows])

        @pl.when(last)
        def _():
            pltpu.sync_copy(zeros_hbm.at[pl.ds(0, TAIL)], acc_sh.at[tail])

    pltpu.sync_copy(src_hbm.at[w], idxs_v)
    pltpu.sync_copy(dst_hbm.at[w], idxd_v)
    plsc.subcore_barrier()

    bufs = (rows0, rows1)
    sems = (sem0, sem1)
    pltpu.async_copy(h_hbm.at[idxs_v.at[0]], rows0, sem0)

    @pl.loop(0, CH, step=2)
    def _pair(j):
        for b in range(2):
            jj = j + b
            nxt = bufs[1 - b]
            nsem = sems[1 - b]

            @pl.when(jj < CH)
            def _():
                @pl.when(jj + 1 < CH)
                def _():
                    pltpu.async_copy(h_hbm.at[idxs_v.at[jj + 1]], nxt, nsem)

                pltpu.make_async_copy(
                    h_hbm.at[idxs_v.at[jj]], bufs[b], sems[b]).wait()
                pltpu.sync_copy(bufs[b], acc_sh.at[idxd_v.at[jj]], add=True)

    plsc.subcore_barrier()
    pltpu.sync_copy(acc_sh.at[my_rows], out_hbm.at[c].at[my_rows])

    @pl.when(last)
    def _():
        pltpu.sync_copy(acc_sh.at[tail], out_hbm.at[c].at[tail])


@functools.cache
def _build_sc_emb():
    mesh = plsc.VectorSubcoreMesh(core_axis_name="c", subcore_axis_name="s")
    return functools.partial(
        pl.kernel,
        out_type=[
            jax.ShapeDtypeStruct((BATCH * NBOX, DREL), jnp.float32),
            jax.ShapeDtypeStruct((BATCH * NBOX, DREL), jnp.float32),
            jax.ShapeDtypeStruct((BATCH * NBOX, HOBJ), jnp.float32),
        ],
        mesh=mesh,
        scratch_types=[
            pltpu.VMEM((LCH, LK), jnp.int32),
            pltpu.VMEM((LK, DREL), jnp.float32),
            pltpu.VMEM((LK, DREL), jnp.float32),
            pltpu.VMEM((LK, HOBJ), jnp.float32),
            pltpu.SemaphoreType.DMA,
        ],
        compiler_params=pltpu.CompilerParams(use_tc_tiling_on_sc=False),
    )(_sc_emb_body)


def _sc_emb(lab, ta, tb, to):
    return _build_sc_emb()(lab, ta, tb, to)


def _sc_emb_body(lab_hbm, ta_hbm, tb_hbm, to_hbm, oa_hbm, ob_hbm, oo_hbm,
                 idx_v, ra, rb, ro, sem):
    c = lax.axis_index("c")
    s = lax.axis_index("s")
    w = c * NS + s
    pltpu.sync_copy(lab_hbm.at[w], idx_v)
    for t in range(LCH):
        base = w * LPW + t * LK
        row = idx_v.at[t]
        pltpu.async_copy(ta_hbm.at[row], ra, sem).wait()
        pltpu.sync_copy(ra, oa_hbm.at[pl.ds(base, LK)])
        pltpu.async_copy(tb_hbm.at[row], rb, sem).wait()
        pltpu.sync_copy(rb, ob_hbm.at[pl.ds(base, LK)])
        pltpu.async_copy(to_hbm.at[row], ro, sem).wait()
        pltpu.sync_copy(ro, oo_hbm.at[pl.ds(base, LK)])


_BLK = 1000
_NBLK = N // _BLK


def _tc_scale_matmul(degp, x, W1):
    def body(deg_ref, x_ref, w_ref, hpa_ref, hpb_ref, dinv_ref):
        d = deg_ref[0, :, 0:1] + deg_ref[1, :, 0:1] + 1.0
        dinv = lax.rsqrt(d)
        h = jnp.dot(x_ref[...], w_ref[...], preferred_element_type=jnp.float32)
        hp = dinv * h
        hpa_ref[...] = hp[:, :HD]
        hpb_ref[...] = hp[:, HD:]
        dinv_ref[...] = jnp.broadcast_to(dinv, (_BLK, D))

    return pl.pallas_call(
        body,
        grid=(_NBLK,),
        in_specs=[
            pl.BlockSpec((NC, _BLK, 16), lambda i: (0, i, 0)),
            pl.BlockSpec((_BLK, D), lambda i: (i, 0)),
            pl.BlockSpec((D, D), lambda i: (0, 0)),
        ],
        out_specs=[
            pl.BlockSpec((_BLK, HD), lambda i: (i, 0)),
            pl.BlockSpec((_BLK, HD), lambda i: (i, 0)),
            pl.BlockSpec((_BLK, D), lambda i: (i, 0)),
        ],
        out_shape=[
            jax.ShapeDtypeStruct((N, HD), jnp.float32),
            jax.ShapeDtypeStruct((N, HD), jnp.float32),
            jax.ShapeDtypeStruct((N, D), jnp.float32),
        ],
    )(degp, x, W1)


def _tc_layer2(acca, accb, dinv, b1, W2):
    def body(acca_ref, accb_ref, dinv_ref, b_ref, w_ref, hpa_ref, hpb_ref):
        acc = jnp.concatenate(
            [acca_ref[0] + acca_ref[1], accb_ref[0] + accb_ref[1]], axis=1)
        z = dinv_ref[...] * acc + b_ref[...]
        z = jnp.maximum(z, 0.0)
        h = jnp.dot(z, w_ref[...], preferred_element_type=jnp.float32)
        hp = dinv_ref[...] * h
        hpa_ref[...] = hp[:, :HD]
        hpb_ref[...] = hp[:, HD:]

    return pl.pallas_call(
        body,
        grid=(_NBLK,),
        in_specs=[
            pl.BlockSpec((NC, _BLK, HD), lambda i: (0, i, 0)),
            pl.BlockSpec((NC, _BLK, HD), lambda i: (0, i, 0)),
            pl.BlockSpec((_BLK, D), lambda i: (i, 0)),
            pl.BlockSpec((1, D), lambda i: (0, 0)),
            pl.BlockSpec((D, D), lambda i: (0, 0)),
        ],
        out_specs=[
            pl.BlockSpec((_BLK, HD), lambda i: (i, 0)),
            pl.BlockSpec((_BLK, HD), lambda i: (i, 0)),
        ],
        out_shape=[
            jax.ShapeDtypeStruct((N, HD), jnp.float32),
            jax.ShapeDtypeStruct((N, HD), jnp.float32),
        ],
    )(acca, accb, dinv, b1, W2)


def _tc_segmean(acca, accb, dinv, b2, batch3):
    def body(acca_ref, accb_ref, dinv_ref, b_ref, bat_ref, out_ref, ssum, cnt):
        i = pl.program_id(0)

        @pl.when(i == 0)
        def _():
            ssum[...] = jnp.zeros_like(ssum)
            cnt[...] = jnp.zeros_like(cnt)

        acc = jnp.concatenate(
            [acca_ref[0] + acca_ref[1], accb_ref[0] + accb_ref[1]], axis=1)
        h2 = dinv_ref[...] * acc + b_ref[...]
        seg = lax.broadcasted_iota(jnp.int32, (BATCH, 1), 0)
        oh = (seg == bat_ref[0]).astype(jnp.float32)
        ssum[...] += jnp.dot(oh, h2, preferred_element_type=jnp.float32)
        cnt[...] += jnp.sum(oh, axis=1, keepdims=True)

        @pl.when(i == _NBLK - 1)
        def _():
            out_ref[...] = ssum[...] / jnp.maximum(cnt[...], 1.0)

    return pl.pallas_call(
        body,
        grid=(_NBLK,),
        in_specs=[
            pl.BlockSpec((NC, _BLK, HD), lambda i: (0, i, 0)),
            pl.BlockSpec((NC, _BLK, HD), lambda i: (0, i, 0)),
            pl.BlockSpec((_BLK, D), lambda i: (i, 0)),
            pl.BlockSpec((1, D), lambda i: (0, 0)),
            pl.BlockSpec((1, 1, _BLK), lambda i: (i, 0, 0)),
        ],
        out_specs=pl.BlockSpec((BATCH, D), lambda i: (0, 0)),
        out_shape=jax.ShapeDtypeStruct((BATCH, D), jnp.float32),
        scratch_shapes=[
            pltpu.VMEM((BATCH, D), jnp.float32),
            pltpu.VMEM((BATCH, 1), jnp.float32),
        ],
    )(acca, accb, dinv, b2, batch3)


_BB = 8


def _tc_obj(ea, eb, eo, cen, cent, b1o, W2o, b2o):
    def body(ea_ref, eb_ref, eo_ref, c_ref, ct_ref, b1_ref, w2_ref, b2_ref,
             out_ref):
        ri = lax.broadcasted_iota(jnp.int32, (PBOX, PBOX), 0)
        ci = lax.broadcasted_iota(jnp.int32, (PBOX, PBOX), 1)
        valid = (ri < NBOX) & (ci < NBOX)
        eye = ri == ci
        for b in range(_BB):
            A = ea_ref[b]
            Bm = eb_ref[b]
            dots = lax.dot_general(A, Bm, (((1,), (1,)), ((), ())),
                                   preferred_element_type=jnp.float32)
            cmat = c_ref[b]
            ctmat = ct_ref[b]
            cc = jnp.dot(cmat, ctmat, preferred_element_type=jnp.float32)
            n2c = jnp.sum(cmat * cmat, axis=1, keepdims=True)
            n2r = jnp.sum(ctmat * ctmat, axis=0, keepdims=True)
            d2 = jnp.maximum(n2c + n2r - 2.0 * cc, 0.0)
            dist = jnp.sqrt(d2 + 1e-9)
            sig = 1.0 / (1.0 + jnp.exp(dist - dots))
            W36 = jnp.where(valid, jnp.where(eye, 1.0, sig), 0.0)
            dego = jnp.maximum(jnp.sum(W36, axis=0, keepdims=True), 1e-6)
            dinvo = lax.rsqrt(dego)
            nW = W36 * jnp.broadcast_to(dinvo, (PBOX, PBOX))
            nW = nW * jnp.sum(
                jnp.where(eye, jnp.broadcast_to(dinvo, (PBOX, PBOX)), 0.0),
                axis=1, keepdims=True)
            g1 = lax.dot_general(nW, eo_ref[b], (((0,), (0,)), ((), ())),
                                 preferred_element_type=jnp.float32)
            g1 = jnp.maximum(g1 + b1_ref[...], 0.0)
            t = jnp.dot(g1, w2_ref[...], preferred_element_type=jnp.float32)
            g2 = lax.dot_general(nW, t, (((0,), (0,)), ((), ())),
                                 preferred_element_type=jnp.float32)
            g2 = g2 + b2_ref[...]
            keep = lax.broadcasted_iota(jnp.int32, (PBOX, D), 0) < NBOX
            out_ref[b, :] = jnp.sum(jnp.where(keep, g2, 0.0), axis=0) / NBOX

    return pl.pallas_call(
        body,
        grid=(BATCH // _BB,),
        in_specs=[
            pl.BlockSpec((_BB, PBOX, DREL), lambda i: (i, 0, 0)),
            pl.BlockSpec((_BB, PBOX, DREL), lambda i: (i, 0, 0)),
            pl.BlockSpec((_BB, PBOX, HOBJ), lambda i: (i, 0, 0)),
            pl.BlockSpec((_BB, PBOX, 2), lambda i: (i, 0, 0)),
            pl.BlockSpec((_BB, 2, PBOX), lambda i: (i, 0, 0)),
            pl.BlockSpec((1, HOBJ), lambda i: (0, 0)),
            pl.BlockSpec((HOBJ, D), lambda i: (0, 0)),
            pl.BlockSpec((1, D), lambda i: (0, 0)),
        ],
        out_specs=pl.BlockSpec((_BB, D), lambda i: (i, 0)),
        out_shape=jax.ShapeDtypeStruct((BATCH, D), jnp.float32),
    )(ea, eb, eo, cen, cent, b1o, W2o, b2o)


_NBLK_F = 5
_FBLK = NANS_P // _NBLK_F


def _tc_fusion(fused, W1p, b1p, W2p, b2p):
    def body(f_ref, w1_ref, b1_ref, w2_ref, b2_ref, out_ref, hdn):
        i = pl.program_id(0)

        @pl.when(i == 0)
        def _():
            hdn[...] = jnp.dot(f_ref[...], w1_ref[...],
                               preferred_element_type=jnp.float32) + b1_ref[...]

        out_ref[...] = jnp.dot(hdn[...], w2_ref[...],
                               preferred_element_type=jnp.float32) + b2_ref[...]

    return pl.pallas_call(
        body,
        grid=(_NBLK_F,),
        in_specs=[
            pl.BlockSpec((BATCH, 2 * D), lambda i: (0, 0)),
            pl.BlockSpec((2 * D, HFUS_P), lambda i: (0, 0)),
            pl.BlockSpec((1, HFUS_P), lambda i: (0, 0)),
            pl.BlockSpec((HFUS_P, _FBLK), lambda i: (0, i)),
            pl.BlockSpec((1, _FBLK), lambda i: (0, i)),
        ],
        out_specs=pl.BlockSpec((BATCH, _FBLK), lambda i: (0, i)),
        out_shape=jax.ShapeDtypeStruct((BATCH, NANS_P), jnp.float32),
        scratch_shapes=[pltpu.VMEM((BATCH, HFUS_P), jnp.float32)],
    )(fused, W1p, b1p, W2p, b2p)


def _tc_logsoftmax(x):
    def body(x_ref, o_ref):
        v = x_ref[...]
        m = jnp.max(v, axis=1, keepdims=True)
        e = jnp.exp(v - m)
        lse = jnp.log(jnp.sum(e, axis=1, keepdims=True)) + m
        o_ref[...] = v - lse

    return pl.pallas_call(
        body,
        out_shape=jax.ShapeDtypeStruct((BATCH, NANS_P), jnp.float32),
    )(x)


def kernel(dep_x, dep_edge_index, dep_batch, boxes, labels,
           W_dep1, b_dep1, W_dep2, b_dep2,
           W_obj1, b_obj1, W_obj2, b_obj2,
           W_rel_a, W_rel_b,
           W_fus1, b_fus1, W_fus2, b_fus2):
    src = dep_edge_index[0].astype(jnp.int32).reshape(NW, CH, K)
    dst = dep_edge_index[1].astype(jnp.int32).reshape(NW, CH, K)

    ones16 = jnp.ones((K, 16), jnp.float32)
    zeros16 = jnp.zeros((RQ, 16), jnp.float32)
    zerosH = jnp.zeros((RQ, HD), jnp.float32)

    degp = _sc_deg(dst, ones16, zeros16)
    hp1a, hp1b, dinv = _tc_scale_matmul(degp, dep_x, W_dep1)
    acc1a = _sc_agg(hp1a, src, dst, zerosH)
    acc1b = _sc_agg(hp1b, src, dst, zerosH)
    hp2a, hp2b = _tc_layer2(acc1a, acc1b, dinv, b_dep1.reshape(1, D), W_dep2)
    acc2a = _sc_agg(hp2a, src, dst, zerosH)
    acc2b = _sc_agg(hp2b, src, dst, zerosH)
    batch3 = dep_batch.astype(jnp.int32).reshape(_NBLK, 1, _BLK)
    dep_out = _tc_segmean(acc2a, acc2b, dinv, b_dep2.reshape(1, D), batch3)

    lab = labels.astype(jnp.int32).reshape(NW, LCH, LK)
    ea, eb, eo = _sc_emb(lab, W_rel_a, W_rel_b, W_obj1)
    pad3 = lambda x: jnp.pad(x.reshape(BATCH, NBOX, -1),
                             ((0, 0), (0, PBOX - NBOX), (0, 0)))
    ea, eb, eo = pad3(ea), pad3(eb), pad3(eo)
    cen = (boxes[..., 0:2] + boxes[..., 2:4]) * 0.5
    cent = jnp.swapaxes(cen, 1, 2)
    cen = jnp.pad(cen, ((0, 0), (0, PBOX - NBOX), (0, 0)))
    cent = jnp.pad(cent, ((0, 0), (0, 0), (0, PBOX - NBOX)))
    obj_out = _tc_obj(ea, eb, eo, cen, cent,
                      b_obj1.reshape(1, HOBJ), W_obj2, b_obj2.reshape(1, D))

    fused = jnp.concatenate([dep_out, obj_out], axis=1)
    W1p = jnp.pad(W_fus1, ((0, 0), (0, HFUS_P - HFUS)))
    b1p = jnp.pad(b_fus1, (0, HFUS_P - HFUS)).reshape(1, HFUS_P)
    W2p = jnp.pad(W_fus2, ((0, HFUS_P - HFUS), (0, NANS_P - NANS)))
    b2p = jnp.pad(b_fus2, (0, NANS_P - NANS),
                  constant_values=-1e30).reshape(1, NANS_P)
    logits = _tc_fusion(fused, W1p, b1p, W2p, b2p)
    out = _tc_logsoftmax(logits)
    return out[:, :NANS]


# speedup vs baseline: 33.0266x; 1.2360x over previous
import functools

import jax
import jax.numpy as jnp
from jax import lax
from jax.experimental import pallas as pl
from jax.experimental.pallas import tpu as pltpu
from jax.experimental.pallas import tpu_sc as plsc

N = 10000
E = 320000
D = 128
BATCH = 128
NBOX = 36
PBOX = 40
NOBJ = 1601
HOBJ = 256
DREL = 64
NANS = 3129
HFUS = (2 * D + NANS) // 2
HFUS_P = 1792
NANS_P = 3200

NC = 2
NS = 16
NW = NC * NS
EPW = E // NW
CH = 125
K = EPW // CH
RQ = 624
TAIL = N - NS * RQ
HD = D // 2
LPW = (BATCH * NBOX) // NW
LCH = 2
LK = LPW // LCH


@functools.cache
def _build_sc_deg():
    mesh = plsc.VectorSubcoreMesh(core_axis_name="c", subcore_axis_name="s")
    return functools.partial(
        pl.kernel,
        out_type=jax.ShapeDtypeStruct((NC, N, 16), jnp.float32),
        mesh=mesh,
        scratch_types=[
            pltpu.VMEM((CH, K), jnp.int32),
            pltpu.VMEM((K, 16), jnp.float32),
            pltpu.VMEM_SHARED((N, 16), jnp.float32),
        ],
        compiler_params=pltpu.CompilerParams(use_tc_tiling_on_sc=False),
    )(_sc_deg_body)


def _sc_deg(dst, ones16, zeros16):
    return _build_sc_deg()(dst, ones16, zeros16)


def _sc_deg_body(dst_hbm, ones_hbm, zeros_hbm, out_hbm, idx_v, ones_v, acc_sh):
    c = lax.axis_index("c")
    s = lax.axis_index("s")
    w = c * NS + s
    my_rows = pl.ds(s * RQ, RQ)
    tail = pl.ds(NS * RQ, TAIL)
    pltpu.sync_copy(zeros_hbm, acc_sh.at[my_rows])

    @pl.when(s == NS - 1)
    def _():
        pltpu.sync_copy(zeros_hbm.at[pl.ds(0, TAIL)], acc_sh.at[tail])

    pltpu.sync_copy(ones_hbm, ones_v)
    pltpu.sync_copy(dst_hbm.at[w], idx_v)
    plsc.subcore_barrier()

    @pl.loop(0, CH)
    def _chunk(j):
        pltpu.sync_copy(ones_v, acc_sh.at[idx_v.at[j]], add=True)

    plsc.subcore_barrier()
    pltpu.sync_copy(acc_sh.at[my_rows], out_hbm.at[c].at[my_rows])

    @pl.when(s == NS - 1)
    def _():
        pltpu.sync_copy(acc_sh.at[tail], out_hbm.at[c].at[tail])


NBUF = 8
PRE = 4


@functools.cache
def _build_sc_agg():
    mesh = plsc.VectorSubcoreMesh(core_axis_name="c", subcore_axis_name="s")
    return functools.partial(
        pl.kernel,
        out_type=[
            jax.ShapeDtypeStruct((NC, N, HD), jnp.float32),
            jax.ShapeDtypeStruct((NC, N, HD), jnp.float32),
        ],
        mesh=mesh,
        scratch_types=(
            [pltpu.VMEM((CH, K), jnp.int32)] * 2
            + [pltpu.VMEM((K, HD), jnp.float32)] * NBUF
            + [pltpu.VMEM_SHARED((N, HD), jnp.float32)]
            + [pltpu.SemaphoreType.DMA] * (2 * NBUF)
        ),
        compiler_params=pltpu.CompilerParams(use_tc_tiling_on_sc=False),
    )(_sc_agg_body)


def _sc_agg(hp_a, hp_b, src, dst, zerosH):
    return _build_sc_agg()(hp_a, hp_b, src, dst, zerosH)


def _sc_agg_body(ha_hbm, hb_hbm, src_hbm, dst_hbm, zeros_hbm,
                 outa_hbm, outb_hbm, *scr):
    idxs_v, idxd_v = scr[0], scr[1]
    bufs = scr[2:2 + NBUF]
    acc_sh = scr[2 + NBUF]
    gsem = scr[3 + NBUF:3 + 2 * NBUF]
    ssem = scr[3 + 2 * NBUF:3 + 3 * NBUF]

    c = lax.axis_index("c")
    si = lax.axis_index("s")
    w = c * NS + si
    my_rows = pl.ds(si * RQ, RQ)
    tail = pl.ds(NS * RQ, TAIL)
    last = si == NS - 1

    pltpu.sync_copy(src_hbm.at[w], idxs_v)
    pltpu.sync_copy(dst_hbm.at[w], idxd_v)

    for h_hbm, out_hbm in ((ha_hbm, outa_hbm), (hb_hbm, outb_hbm)):
        @pl.when(c == 0)
        def _():
            pltpu.sync_copy(h_hbm.at[my_rows], acc_sh.at[my_rows])

            @pl.when(last)
            def _():
                pltpu.sync_copy(h_hbm.at[tail], acc_sh.at[tail])

        @pl.when(c != 0)
        def _():
            pltpu.sync_copy(zeros_hbm, acc_sh.at[my_rows])

            @pl.when(last)
            def _():
                pltpu.sync_copy(zeros_hbm.at[pl.ds(0, TAIL)], acc_sh.at[tail])

        plsc.subcore_barrier()

        for p in range(PRE):
            pltpu.async_copy(h_hbm.at[idxs_v.at[p]], bufs[p], gsem[p])

        @pl.loop(0, CH, step=NBUF)
        def _ring(j):
            for b in range(NBUF):
                jj = j + b

                @pl.when(jj < CH)
                def _():
                    pltpu.make_async_copy(
                        h_hbm.at[idxs_v.at[jj]], bufs[b], gsem[b]).wait()
                    pltpu.async_copy(
                        bufs[b], acc_sh.at[idxd_v.at[jj]], ssem[b], add=True)
                    pre = jj + PRE
                    bp = (b + PRE) % NBUF

                    @pl.when(pre < CH)
                    def _():
                        @pl.when(pre >= NBUF)
                        def _():
                            pltpu.make_async_copy(
                                bufs[bp], acc_sh.at[idxd_v.at[pre - NBUF]],
                                ssem[bp]).wait()

                        pltpu.async_copy(
                            h_hbm.at[idxs_v.at[pre]], bufs[bp], gsem[bp])

        for b in range(NBUF):
            xb = (CH - NBUF) + ((b - (CH - NBUF)) % NBUF)
            pltpu.make_async_copy(
                bufs[b], acc_sh.at[idxd_v.at[xb]], ssem[b]).wait()

        plsc.subcore_barrier()
        pltpu.sync_copy(acc_sh.at[my_rows], out_hbm.at[c].at[my_rows])

        @pl.when(last)
        def _():
            pltpu.sync_copy(acc_sh.at[tail], out_hbm.at[c].at[tail])

        plsc.subcore_barrier()


@functools.cache
def _build_sc_emb():
    mesh = plsc.VectorSubcoreMesh(core_axis_name="c", subcore_axis_name="s")
    return functools.partial(
        pl.kernel,
        out_type=[
            jax.ShapeDtypeStruct((BATCH * NBOX, DREL), jnp.float32),
            jax.ShapeDtypeStruct((BATCH * NBOX, DREL), jnp.float32),
            jax.ShapeDtypeStruct((BATCH * NBOX, HOBJ), jnp.float32),
        ],
        mesh=mesh,
        scratch_types=[
            pltpu.VMEM((LCH, LK), jnp.int32),
            pltpu.VMEM((LK, DREL), jnp.float32),
            pltpu.VMEM((LK, DREL), jnp.float32),
            pltpu.VMEM((LK, HOBJ), jnp.float32),
            pltpu.SemaphoreType.DMA,
        ],
        compiler_params=pltpu.CompilerParams(use_tc_tiling_on_sc=False),
    )(_sc_emb_body)


def _sc_emb(lab, ta, tb, to):
    return _build_sc_emb()(lab, ta, tb, to)


def _sc_emb_body(lab_hbm, ta_hbm, tb_hbm, to_hbm, oa_hbm, ob_hbm, oo_hbm,
                 idx_v, ra, rb, ro, sem):
    c = lax.axis_index("c")
    s = lax.axis_index("s")
    w = c * NS + s
    pltpu.sync_copy(lab_hbm.at[w], idx_v)
    for t in range(LCH):
        base = w * LPW + t * LK
        row = idx_v.at[t]
        pltpu.async_copy(ta_hbm.at[row], ra, sem).wait()
        pltpu.sync_copy(ra, oa_hbm.at[pl.ds(base, LK)])
        pltpu.async_copy(tb_hbm.at[row], rb, sem).wait()
        pltpu.sync_copy(rb, ob_hbm.at[pl.ds(base, LK)])
        pltpu.async_copy(to_hbm.at[row], ro, sem).wait()
        pltpu.sync_copy(ro, oo_hbm.at[pl.ds(base, LK)])


_BLK = 1000
_NBLK = N // _BLK


def _tc_scale_matmul(degp, x, W1):
    def body(deg_ref, x_ref, w_ref, hpa_ref, hpb_ref, dinv_ref):
        d = deg_ref[0, :, 0:1] + deg_ref[1, :, 0:1] + 1.0
        dinv = lax.rsqrt(d)
        h = jnp.dot(x_ref[...], w_ref[...], preferred_element_type=jnp.float32)
        hp = dinv * h
        hpa_ref[...] = hp[:, :HD]
        hpb_ref[...] = hp[:, HD:]
        dinv_ref[...] = jnp.broadcast_to(dinv, (_BLK, D))

    return pl.pallas_call(
        body,
        grid=(_NBLK,),
        in_specs=[
            pl.BlockSpec((NC, _BLK, 16), lambda i: (0, i, 0)),
            pl.BlockSpec((_BLK, D), lambda i: (i, 0)),
            pl.BlockSpec((D, D), lambda i: (0, 0)),
        ],
        out_specs=[
            pl.BlockSpec((_BLK, HD), lambda i: (i, 0)),
            pl.BlockSpec((_BLK, HD), lambda i: (i, 0)),
            pl.BlockSpec((_BLK, D), lambda i: (i, 0)),
        ],
        out_shape=[
            jax.ShapeDtypeStruct((N, HD), jnp.float32),
            jax.ShapeDtypeStruct((N, HD), jnp.float32),
            jax.ShapeDtypeStruct((N, D), jnp.float32),
        ],
    )(degp, x, W1)


def _tc_layer2(acca, accb, dinv, b1, W2):
    def body(acca_ref, accb_ref, dinv_ref, b_ref, w_ref, hpa_ref, hpb_ref):
        acc = jnp.concatenate(
            [acca_ref[0] + acca_ref[1], accb_ref[0] + accb_ref[1]], axis=1)
        z = dinv_ref[...] * acc + b_ref[...]
        z = jnp.maximum(z, 0.0)
        h = jnp.dot(z, w_ref[...], preferred_element_type=jnp.float32)
        hp = dinv_ref[...] * h
        hpa_ref[...] = hp[:, :HD]
        hpb_ref[...] = hp[:, HD:]

    return pl.pallas_call(
        body,
        grid=(_NBLK,),
        in_specs=[
            pl.BlockSpec((NC, _BLK, HD), lambda i: (0, i, 0)),
            pl.BlockSpec((NC, _BLK, HD), lambda i: (0, i, 0)),
            pl.BlockSpec((_BLK, D), lambda i: (i, 0)),
            pl.BlockSpec((1, D), lambda i: (0, 0)),
            pl.BlockSpec((D, D), lambda i: (0, 0)),
        ],
        out_specs=[
            pl.BlockSpec((_BLK, HD), lambda i: (i, 0)),
            pl.BlockSpec((_BLK, HD), lambda i: (i, 0)),
        ],
        out_shape=[
            jax.ShapeDtypeStruct((N, HD), jnp.float32),
            jax.ShapeDtypeStruct((N, HD), jnp.float32),
        ],
    )(acca, accb, dinv, b1, W2)


def _tc_segmean(acca, accb, dinv, b2, batch3):
    def body(acca_ref, accb_ref, dinv_ref, b_ref, bat_ref, out_ref, ssum, cnt):
        i = pl.program_id(0)

        @pl.when(i == 0)
        def _():
            ssum[...] = jnp.zeros_like(ssum)
            cnt[...] = jnp.zeros_like(cnt)

        acc = jnp.concatenate(
            [acca_ref[0] + acca_ref[1], accb_ref[0] + accb_ref[1]], axis=1)
        h2 = dinv_ref[...] * acc + b_ref[...]
        seg = lax.broadcasted_iota(jnp.int32, (BATCH, 1), 0)
        oh = (seg == bat_ref[0]).astype(jnp.float32)
        ssum[...] += jnp.dot(oh, h2, preferred_element_type=jnp.float32)
        cnt[...] += jnp.sum(oh, axis=1, keepdims=True)

        @pl.when(i == _NBLK - 1)
        def _():
            out_ref[...] = ssum[...] / jnp.maximum(cnt[...], 1.0)

    return pl.pallas_call(
        body,
        grid=(_NBLK,),
        in_specs=[
            pl.BlockSpec((NC, _BLK, HD), lambda i: (0, i, 0)),
            pl.BlockSpec((NC, _BLK, HD), lambda i: (0, i, 0)),
            pl.BlockSpec((_BLK, D), lambda i: (i, 0)),
            pl.BlockSpec((1, D), lambda i: (0, 0)),
            pl.BlockSpec((1, 1, _BLK), lambda i: (i, 0, 0)),
        ],
        out_specs=pl.BlockSpec((BATCH, D), lambda i: (0, 0)),
        out_shape=jax.ShapeDtypeStruct((BATCH, D), jnp.float32),
        scratch_shapes=[
            pltpu.VMEM((BATCH, D), jnp.float32),
            pltpu.VMEM((BATCH, 1), jnp.float32),
        ],
    )(acca, accb, dinv, b2, batch3)


_BB = 8


def _tc_obj(ea, eb, eo, cen, cent, b1o, W2o, b2o):
    def body(ea_ref, eb_ref, eo_ref, c_ref, ct_ref, b1_ref, w2_ref, b2_ref,
             out_ref):
        ri = lax.broadcasted_iota(jnp.int32, (PBOX, PBOX), 0)
        ci = lax.broadcasted_iota(jnp.int32, (PBOX, PBOX), 1)
        valid = (ri < NBOX) & (ci < NBOX)
        eye = ri == ci
        for b in range(_BB):
            A = ea_ref[b]
            Bm = eb_ref[b]
            dots = lax.dot_general(A, Bm, (((1,), (1,)), ((), ())),
                                   preferred_element_type=jnp.float32)
            cmat = c_ref[b]
            ctmat = ct_ref[b]
            cc = jnp.dot(cmat, ctmat, preferred_element_type=jnp.float32)
            n2c = jnp.sum(cmat * cmat, axis=1, keepdims=True)
            n2r = jnp.sum(ctmat * ctmat, axis=0, keepdims=True)
            d2 = jnp.maximum(n2c + n2r - 2.0 * cc, 0.0)
            dist = jnp.sqrt(d2 + 1e-9)
            sig = 1.0 / (1.0 + jnp.exp(dist - dots))
            W36 = jnp.where(valid, jnp.where(eye, 1.0, sig), 0.0)
            dego = jnp.maximum(jnp.sum(W36, axis=0, keepdims=True), 1e-6)
            dinvo = lax.rsqrt(dego)
            nW = W36 * jnp.broadcast_to(dinvo, (PBOX, PBOX))
            nW = nW * jnp.sum(
                jnp.where(eye, jnp.broadcast_to(dinvo, (PBOX, PBOX)), 0.0),
                axis=1, keepdims=True)
            g1 = lax.dot_general(nW, eo_ref[b], (((0,), (0,)), ((), ())),
                                 preferred_element_type=jnp.float32)
            g1 = jnp.maximum(g1 + b1_ref[...], 0.0)
            t = jnp.dot(g1, w2_ref[...], preferred_element_type=jnp.float32)
            g2 = lax.dot_general(nW, t, (((0,), (0,)), ((), ())),
                                 preferred_element_type=jnp.float32)
            g2 = g2 + b2_ref[...]
            keep = lax.broadcasted_iota(jnp.int32, (PBOX, D), 0) < NBOX
            out_ref[b, :] = jnp.sum(jnp.where(keep, g2, 0.0), axis=0) / NBOX

    return pl.pallas_call(
        body,
        grid=(BATCH // _BB,),
        in_specs=[
            pl.BlockSpec((_BB, PBOX, DREL), lambda i: (i, 0, 0)),
            pl.BlockSpec((_BB, PBOX, DREL), lambda i: (i, 0, 0)),
            pl.BlockSpec((_BB, PBOX, HOBJ), lambda i: (i, 0, 0)),
            pl.BlockSpec((_BB, PBOX, 2), lambda i: (i, 0, 0)),
            pl.BlockSpec((_BB, 2, PBOX), lambda i: (i, 0, 0)),
            pl.BlockSpec((1, HOBJ), lambda i: (0, 0)),
            pl.BlockSpec((HOBJ, D), lambda i: (0, 0)),
            pl.BlockSpec((1, D), lambda i: (0, 0)),
        ],
        out_specs=pl.BlockSpec((_BB, D), lambda i: (i, 0)),
        out_shape=jax.ShapeDtypeStruct((BATCH, D), jnp.float32),
    )(ea, eb, eo, cen, cent, b1o, W2o, b2o)


_NBLK_F = 5
_FBLK = NANS_P // _NBLK_F


def _tc_fusion(fused, W1p, b1p, W2p, b2p):
    def body(f_ref, w1_ref, b1_ref, w2_ref, b2_ref, out_ref, hdn):
        i = pl.program_id(0)

        @pl.when(i == 0)
        def _():
            hdn[...] = jnp.dot(f_ref[...], w1_ref[...],
                               preferred_element_type=jnp.float32) + b1_ref[...]

        out_ref[...] = jnp.dot(hdn[...], w2_ref[...],
                               preferred_element_type=jnp.float32) + b2_ref[...]

    return pl.pallas_call(
        body,
        grid=(_NBLK_F,),
        in_specs=[
            pl.BlockSpec((BATCH, 2 * D), lambda i: (0, 0)),
            pl.BlockSpec((2 * D, HFUS_P), lambda i: (0, 0)),
            pl.BlockSpec((1, HFUS_P), lambda i: (0, 0)),
            pl.BlockSpec((HFUS_P, _FBLK), lambda i: (0, i)),
            pl.BlockSpec((1, _FBLK), lambda i: (0, i)),
        ],
        out_specs=pl.BlockSpec((BATCH, _FBLK), lambda i: (0, i)),
        out_shape=jax.ShapeDtypeStruct((BATCH, NANS_P), jnp.float32),
        scratch_shapes=[pltpu.VMEM((BATCH, HFUS_P), jnp.float32)],
    )(fused, W1p, b1p, W2p, b2p)


def _tc_logsoftmax(x):
    def body(x_ref, o_ref):
        v = x_ref[...]
        m = jnp.max(v, axis=1, keepdims=True)
        e = jnp.exp(v - m)
        lse = jnp.log(jnp.sum(e, axis=1, keepdims=True)) + m
        o_ref[...] = v - lse

    return pl.pallas_call(
        body,
        out_shape=jax.ShapeDtypeStruct((BATCH, NANS_P), jnp.float32),
    )(x)


def kernel(dep_x, dep_edge_index, dep_batch, boxes, labels,
           W_dep1, b_dep1, W_dep2, b_dep2,
           W_obj1, b_obj1, W_obj2, b_obj2,
           W_rel_a, W_rel_b,
           W_fus1, b_fus1, W_fus2, b_fus2):
    src = dep_edge_index[0].astype(jnp.int32).reshape(NW, CH, K)
    dst = dep_edge_index[1].astype(jnp.int32).reshape(NW, CH, K)

    ones16 = jnp.ones((K, 16), jnp.float32)
    zeros16 = jnp.zeros((RQ, 16), jnp.float32)
    zerosH = jnp.zeros((RQ, HD), jnp.float32)

    degp = _sc_deg(dst, ones16, zeros16)
    hp1a, hp1b, dinv = _tc_scale_matmul(degp, dep_x, W_dep1)
    acc1a, acc1b = _sc_agg(hp1a, hp1b, src, dst, zerosH)
    hp2a, hp2b = _tc_layer2(acc1a, acc1b, dinv, b_dep1.reshape(1, D), W_dep2)
    acc2a, acc2b = _sc_agg(hp2a, hp2b, src, dst, zerosH)
    batch3 = dep_batch.astype(jnp.int32).reshape(_NBLK, 1, _BLK)
    dep_out = _tc_segmean(acc2a, acc2b, dinv, b_dep2.reshape(1, D), batch3)

    lab = labels.astype(jnp.int32).reshape(NW, LCH, LK)
    ea, eb, eo = _sc_emb(lab, W_rel_a, W_rel_b, W_obj1)
    pad3 = lambda x: jnp.pad(x.reshape(BATCH, NBOX, -1),
                             ((0, 0), (0, PBOX - NBOX), (0, 0)))
    ea, eb, eo = pad3(ea), pad3(eb), pad3(eo)
    cen = (boxes[..., 0:2] + boxes[..., 2:4]) * 0.5
    cent = jnp.swapaxes(cen, 1, 2)
    cen = jnp.pad(cen, ((0, 0), (0, PBOX - NBOX), (0, 0)))
    cent = jnp.pad(cent, ((0, 0), (0, 0), (0, PBOX - NBOX)))
    obj_out = _tc_obj(ea, eb, eo, cen, cent,
                      b_obj1.reshape(1, HOBJ), W_obj2, b_obj2.reshape(1, D))

    fused = jnp.concatenate([dep_out, obj_out], axis=1)
    W1p = jnp.pad(W_fus1, ((0, 0), (0, HFUS_P - HFUS)))
    b1p = jnp.pad(b_fus1, (0, HFUS_P - HFUS)).reshape(1, HFUS_P)
    W2p = jnp.pad(W_fus2, ((0, HFUS_P - HFUS), (0, NANS_P - NANS)))
    b2p = jnp.pad(b_fus2, (0, NANS_P - NANS),
                  constant_values=-1e30).reshape(1, NANS_P)
    logits = _tc_fusion(fused, W1p, b1p, W2p, b2p)
    out = _tc_logsoftmax(logits)
    return out[:, :NANS]
